# baseline (speedup 1.0000x reference)
import functools

import jax
import jax.numpy as jnp
from jax import lax
from jax.experimental import pallas as pl
from jax.experimental.pallas import tpu as pltpu
from jax.experimental.pallas import tpu_sc as plsc

VOCAB = 1000000
EMBED_DIM = 16
BATCH = 16384
N_FIELDS = 26

_NUM_IDX = BATCH * N_FIELDS
_NW = 32
_PER_W = _NUM_IDX // _NW
_CHUNK = 1664
_NCHUNK = _PER_W // _CHUNK
_UNROLL = 8


def _sig_kernel(table_hbm, idx_hbm, out_hbm, idx_v, rows_v, sem):
    wid = lax.axis_index("s") * 2 + lax.axis_index("c")
    base = wid * _PER_W
    for c in range(_NCHUNK):
        start = base + c * _CHUNK
        pltpu.sync_copy(idx_hbm.at[pl.ds(start, _CHUNK)], idx_v)
        pltpu.async_copy(table_hbm.at[idx_v], rows_v, sem).wait()

        def body(j, carry):
            for u in range(_UNROLL):
                i = j * _UNROLL + u
                r = rows_v[i]
                rows_v[i] = 1.0 / (1.0 + jnp.exp(-r))
            return carry

        lax.fori_loop(0, _CHUNK // _UNROLL, body, 0)
        pltpu.sync_copy(rows_v, out_hbm.at[pl.ds(start, _CHUNK)])


@jax.jit
def _run(table, xf):
    mesh = plsc.VectorSubcoreMesh(core_axis_name="c", subcore_axis_name="s")
    f = functools.partial(
        pl.kernel,
        mesh=mesh,
        out_type=jax.ShapeDtypeStruct((_NUM_IDX, EMBED_DIM), jnp.float32),
        scratch_types=[
            pltpu.VMEM((_CHUNK,), jnp.int32),
            pltpu.VMEM((_CHUNK, EMBED_DIM), jnp.float32),
            pltpu.SemaphoreType.DMA,
        ],
        compiler_params=pltpu.CompilerParams(use_tc_tiling_on_sc=False),
    )(_sig_kernel)
    return f(table, xf)


def kernel(table, x):
    out = _run(table, x.reshape(-1))
    return out.reshape(BATCH, N_FIELDS, EMBED_DIM)
